# Initial kernel scaffold; baseline (speedup 1.0000x reference)
import jax
import jax.numpy as jnp
from jax.experimental import pallas as pl


def kernel(L, rel_pos_embed):
    raise NotImplementedError("write your pallas kernel here")


# speedup vs baseline: 2.7546x; 2.7546x over previous
import jax
import jax.numpy as jnp
from jax import lax
from jax.experimental import pallas as pl
from jax.experimental.pallas import tpu as pltpu
from jax.experimental.pallas import tpu_sc as plsc

L_OUT = 384
D = 256
ROWS_PER_WORKER = 12
WIN_ROWS = L_OUT + ROWS_PER_WORKER - 1
ROW_W = L_OUT * D


def _pairwise_body(table_hbm, out_hbm, win, sem):
    c = lax.axis_index("c")
    s = lax.axis_index("s")
    wid = s * 2 + c
    a = wid * ROWS_PER_WORKER
    pltpu.sync_copy(table_hbm.at[pl.ds((489 - a) * D, WIN_ROWS * D)], win)
    copies = []
    for r in range(ROWS_PER_WORKER):
        copies.append(
            pltpu.async_copy(
                win.at[pl.ds((ROWS_PER_WORKER - 1 - r) * D, ROW_W)],
                out_hbm.at[pl.ds((a + r) * ROW_W, ROW_W)],
                sem,
            )
        )
    for cp in copies:
        cp.wait()


def kernel(L, rel_pos_embed):
    mesh = plsc.VectorSubcoreMesh(core_axis_name="c", subcore_axis_name="s")
    run = pl.kernel(
        _pairwise_body,
        out_type=jax.ShapeDtypeStruct((L_OUT * L_OUT * D,), jnp.float32),
        mesh=mesh,
        scratch_types=[
            pltpu.VMEM((WIN_ROWS * D,), jnp.float32),
            pltpu.SemaphoreType.DMA,
        ],
    )
    flat = run(rel_pos_embed.reshape(-1))
    return flat.reshape(L_OUT, L_OUT, D)
